# baseline (speedup 1.0000x reference)
import functools

import jax
import jax.numpy as jnp
from jax import lax
from jax.experimental import pallas as pl
from jax.experimental.pallas import tpu as pltpu
from jax.experimental.pallas import tpu_sc as plsc

EMBED_DIM = 64
NUM_CORES = 2
NUM_SUBCORES = 16
NUM_WORKERS = NUM_CORES * NUM_SUBCORES
CHUNK = 320
N_BUF = 4


@functools.partial(jax.jit, static_argnames=("b_per_w", "n_chunks"))
def _gather(idx, table, *, b_per_w, n_chunks):
  B = idx.shape[0]
  mesh = plsc.VectorSubcoreMesh(core_axis_name="c", subcore_axis_name="s")

  @functools.partial(
      pl.kernel,
      out_type=jax.ShapeDtypeStruct((B, EMBED_DIM), jnp.float32),
      mesh=mesh,
      scratch_types=[
          pltpu.VMEM((b_per_w,), jnp.int32),
          pltpu.VMEM((CHUNK, EMBED_DIM), jnp.float32),
          pltpu.VMEM((CHUNK, EMBED_DIM), jnp.float32),
          pltpu.VMEM((CHUNK, EMBED_DIM), jnp.float32),
          pltpu.VMEM((CHUNK, EMBED_DIM), jnp.float32),
          pltpu.SemaphoreType.DMA,
          pltpu.SemaphoreType.DMA,
          pltpu.SemaphoreType.DMA,
          pltpu.SemaphoreType.DMA,
      ],
      compiler_params=pltpu.CompilerParams(use_tc_tiling_on_sc=False),
  )
  def k(idx_hbm, table_hbm, out_hbm, idx_v, rows0, rows1, rows2, rows3,
        sem0, sem1, sem2, sem3):
    wid = lax.axis_index("s") * NUM_CORES + lax.axis_index("c")
    base = wid * b_per_w
    rows = (rows0, rows1, rows2, rows3)
    sems = (sem0, sem1, sem2, sem3)

    pltpu.sync_copy(idx_hbm.at[pl.ds(base, b_per_w)], idx_v)

    def gather(c, b):
      return pltpu.make_async_copy(
          table_hbm.at[idx_v.at[pl.ds(c * CHUNK, CHUNK)]], rows[b], sems[b])

    for b in range(N_BUF):
      gather(b, b).start()

    @pl.loop(0, n_chunks, step=N_BUF)
    def _(g):
      for b in range(N_BUF):
        c = g + b
        gather(c, b).wait()
        pltpu.sync_copy(rows[b], out_hbm.at[pl.ds(base + c * CHUNK, CHUNK)])
        nxt = c + N_BUF

        @pl.when(nxt < n_chunks)
        def _():
          gather(nxt, b).start()

  return k(idx, table)


def kernel(x, table):
  B = x.shape[0] * x.shape[1]
  b_per_w = B // NUM_WORKERS
  n_chunks = b_per_w // CHUNK
  idx = x.reshape(B).astype(jnp.int32)
  out = _gather(idx, table, b_per_w=b_per_w, n_chunks=n_chunks)
  return out.reshape(x.shape[0], x.shape[1], EMBED_DIM)


# speedup vs baseline: 1.2497x; 1.2497x over previous
import functools

import jax
import jax.numpy as jnp
from jax import lax
from jax.experimental import pallas as pl
from jax.experimental.pallas import tpu as pltpu
from jax.experimental.pallas import tpu_sc as plsc

EMBED_DIM = 64
PAD_DIM = 128
HIST_PAD = 56
NUM_CORES = 2
NUM_SUBCORES = 16
NUM_WORKERS = NUM_CORES * NUM_SUBCORES
CHUNK_B = 8
N_BUF = 2


@functools.partial(jax.jit, static_argnames=("batch", "hist", "b_per_w"))
def _gather(idx, padded, *, batch, hist, b_per_w):
  mesh = plsc.VectorSubcoreMesh(core_axis_name="c", subcore_axis_name="s")
  rows_per_chunk = CHUNK_B * hist
  n_chunks = b_per_w // CHUNK_B

  @functools.partial(
      pl.kernel,
      out_type=jax.ShapeDtypeStruct((batch, HIST_PAD, PAD_DIM), jnp.float32),
      mesh=mesh,
      scratch_types=[
          pltpu.VMEM((b_per_w * hist,), jnp.int32),
          pltpu.VMEM((rows_per_chunk, PAD_DIM), jnp.float32),
          pltpu.VMEM((rows_per_chunk, PAD_DIM), jnp.float32),
          pltpu.SemaphoreType.DMA,
          pltpu.SemaphoreType.DMA,
      ],
      compiler_params=pltpu.CompilerParams(use_tc_tiling_on_sc=False),
  )
  def k(idx_hbm, table_hbm, out_hbm, idx_v, rows0, rows1, sem0, sem1):
    wid = lax.axis_index("s") * NUM_CORES + lax.axis_index("c")
    base_b = wid * b_per_w
    rows = (rows0, rows1)
    sems = (sem0, sem1)

    pltpu.sync_copy(idx_hbm.at[pl.ds(base_b * hist, b_per_w * hist)], idx_v)

    def gather(c, b):
      return pltpu.make_async_copy(
          table_hbm.at[idx_v.at[pl.ds(c * rows_per_chunk, rows_per_chunk)]],
          rows[b], sems[b])

    for b in range(N_BUF):
      gather(b, b).start()

    @pl.loop(0, n_chunks, step=N_BUF)
    def _(g):
      for b in range(N_BUF):
        c = g + b
        gather(c, b).wait()
        for j in range(CHUNK_B):
          pltpu.sync_copy(
              rows[b].at[pl.ds(j * hist, hist), :],
              out_hbm.at[base_b + c * CHUNK_B + j, pl.ds(0, hist), :])
        nxt = c + N_BUF

        @pl.when(nxt < n_chunks)
        def _():
          gather(nxt, b).start()

  return k(idx, padded)


def kernel(x, table):
  batch, hist = x.shape
  b_per_w = batch // NUM_WORKERS
  idx = x.reshape(batch * hist).astype(jnp.int32)
  padded = jnp.pad(table, ((0, 0), (0, PAD_DIM - EMBED_DIM)))
  out = _gather(idx, padded, batch=batch, hist=hist, b_per_w=b_per_w)
  return out[:, :hist, :EMBED_DIM]
